# Initial kernel scaffold; baseline (speedup 1.0000x reference)
import jax
import jax.numpy as jnp
from jax.experimental import pallas as pl


def kernel(x, edge_index, params):
    raise NotImplementedError("write your pallas kernel here")


# speedup vs baseline: 2.2296x; 2.2296x over previous
import functools

import jax
import jax.numpy as jnp
from jax import lax
from jax.experimental import pallas as pl
from jax.experimental.pallas import tpu as pltpu
from jax.experimental.pallas import tpu_sc as plsc

N_NODES = 10000
N_EDGES = 320000
F_IN = 128
BIG = 128
HID = 32
EPS = 1e-5

NC = 2
NS = 16
NW = NC * NS
CHUNK = 80
EPW = N_EDGES // NW
NCHUNK = EPW // CHUNK
N_PAD = 10240
ROWS_PER_SUB = N_PAD // NS

_PREC = lax.Precision.HIGHEST


def _sds(shape, dtype=jnp.float32):
    return jax.ShapeDtypeStruct(shape, dtype)


def _vmesh():
    return plsc.VectorSubcoreMesh(core_axis_name="c", subcore_axis_name="s")


@functools.lru_cache(maxsize=None)
def _make_sc_gather():
    @functools.partial(
        pl.kernel,
        out_type=(_sds((N_EDGES, BIG)), _sds((N_EDGES, BIG))),
        mesh=_vmesh(),
        scratch_types=[
            pltpu.VMEM((EPW,), jnp.int32),
            pltpu.VMEM((EPW,), jnp.int32),
            pltpu.VMEM((CHUNK, BIG), jnp.float32),
            pltpu.VMEM((CHUNK, BIG), jnp.float32),
        ],
    )
    def _sc_gather(ta_hbm, tb_hbm, src_hbm, dst_hbm, g1_hbm, g2_hbm,
                   idxs_v, idxd_v, bufa, bufb):
        w = lax.axis_index("s") * NC + lax.axis_index("c")
        base = w * EPW
        pltpu.sync_copy(src_hbm.at[pl.ds(base, EPW)], idxs_v)
        pltpu.sync_copy(dst_hbm.at[pl.ds(base, EPW)], idxd_v)

        @pl.loop(0, EPW, step=CHUNK)
        def _(off):
            pltpu.sync_copy(ta_hbm.at[idxd_v.at[pl.ds(off, CHUNK)]], bufa)
            pltpu.sync_copy(tb_hbm.at[idxs_v.at[pl.ds(off, CHUNK)]], bufb)
            pltpu.sync_copy(bufa, g1_hbm.at[pl.ds(base + off, CHUNK)])
            pltpu.sync_copy(bufb, g2_hbm.at[pl.ds(base + off, CHUNK)])

    return _sc_gather


@functools.lru_cache(maxsize=None)
def _make_sc_scatter(d):
    @functools.partial(
        pl.kernel,
        out_type=_sds((NC, N_PAD, d)),
        mesh=_vmesh(),
        scratch_types=[
            pltpu.VMEM((NCHUNK, CHUNK), jnp.int32),
            pltpu.VMEM((CHUNK, d), jnp.float32),
            pltpu.VMEM_SHARED((N_PAD, d), jnp.float32),
        ],
    )
    def scatter_k(h_hbm, dst3_hbm, zeros_hbm, out_hbm, idx_v, buf, acc):
        c = lax.axis_index("c")
        s = lax.axis_index("s")
        pltpu.sync_copy(zeros_hbm.at[pl.ds(s * ROWS_PER_SUB, ROWS_PER_SUB)],
                        acc.at[pl.ds(s * ROWS_PER_SUB, ROWS_PER_SUB)])
        w = c * NS + s
        base = w * EPW
        pltpu.sync_copy(dst3_hbm.at[w], idx_v)
        plsc.subcore_barrier()

        @pl.loop(0, NCHUNK)
        def _(j):
            pltpu.sync_copy(h_hbm.at[pl.ds(base + j * CHUNK, CHUNK)], buf)
            pltpu.sync_copy(buf, acc.at[idx_v.at[j]], add=True)

        plsc.subcore_barrier()
        pltpu.sync_copy(acc.at[pl.ds(s * ROWS_PER_SUB, ROWS_PER_SUB)],
                        out_hbm.at[c, pl.ds(s * ROWS_PER_SUB, ROWS_PER_SUB)])

    return scatter_k


@functools.lru_cache(maxsize=None)
def _make_sc_counts():
    @functools.partial(
        pl.kernel,
        out_type=_sds((NC, N_PAD, BIG)),
        mesh=_vmesh(),
        scratch_types=[
            pltpu.VMEM((NCHUNK, CHUNK), jnp.int32),
            pltpu.VMEM((CHUNK, BIG), jnp.float32),
            pltpu.VMEM_SHARED((N_PAD, BIG), jnp.float32),
        ],
    )
    def _sc_counts(dst3_hbm, ones_hbm, zeros_hbm, out_hbm, idx_v, buf, acc):
        c = lax.axis_index("c")
        s = lax.axis_index("s")
        pltpu.sync_copy(zeros_hbm.at[pl.ds(s * ROWS_PER_SUB, ROWS_PER_SUB)],
                        acc.at[pl.ds(s * ROWS_PER_SUB, ROWS_PER_SUB)])
        w = c * NS + s
        pltpu.sync_copy(dst3_hbm.at[w], idx_v)
        pltpu.sync_copy(ones_hbm, buf)
        plsc.subcore_barrier()

        @pl.loop(0, NCHUNK)
        def _(j):
            pltpu.sync_copy(buf, acc.at[idx_v.at[j]], add=True)

        plsc.subcore_barrier()
        pltpu.sync_copy(acc.at[pl.ds(s * ROWS_PER_SUB, ROWS_PER_SUB)],
                        out_hbm.at[c, pl.ds(s * ROWS_PER_SUB, ROWS_PER_SUB)])

    return _sc_counts


def _bn_node_body(x_ref, g_ref, be_ref, wd_ref, ws_ref, b1_ref,
                  ta_ref, tb_ref):
    x = x_ref[...]
    mu = jnp.mean(x, axis=0, keepdims=True)
    var = jnp.mean((x - mu) ** 2, axis=0, keepdims=True)
    y = (x - mu) * (1.0 / jnp.sqrt(var + EPS)) * g_ref[...] + be_ref[...]
    ta_ref[...] = jnp.dot(y, wd_ref[...], precision=_PREC) + b1_ref[...]
    tb_ref[...] = jnp.dot(y, ws_ref[...], precision=_PREC)


def _bn_node(x, gamma, beta, wd, ws, b1):
    return pl.pallas_call(
        _bn_node_body,
        out_shape=(_sds((N_NODES, BIG)), _sds((N_NODES, BIG))),
    )(x, gamma, beta, wd, ws, b1)


def _prep_cnt_body(c_ref, inv_ref, msk_ref):
    cnt = c_ref[0, :N_NODES, 0:1] + c_ref[1, :N_NODES, 0:1]
    inv_ref[...] = 1.0 / jnp.maximum(cnt, 1.0)
    msk_ref[...] = jnp.where(cnt > 0.0, 1.0, 0.0)


def _prep_cnt(cnts):
    return pl.pallas_call(
        _prep_cnt_body,
        out_shape=(_sds((N_NODES, 1)), _sds((N_NODES, 1))),
    )(cnts)


def _comb_node_body(s_ref, inv_ref, wd_ref, ws_ref, b1_ref, ta_ref, tb_ref):
    f = wd_ref.shape[0]
    y = (s_ref[0, :N_NODES, :f] + s_ref[1, :N_NODES, :f]) * inv_ref[...]
    ta_ref[...] = jnp.dot(y, wd_ref[...], precision=_PREC) + b1_ref[...]
    tb_ref[...] = jnp.dot(y, ws_ref[...], precision=_PREC)


def _comb_node(seg, inv, wd, ws, b1):
    return pl.pallas_call(
        _comb_node_body,
        out_shape=(_sds((N_NODES, BIG)), _sds((N_NODES, BIG))),
    )(seg, inv, wd, ws, b1)


def _edge_mlp_body(g1_ref, g2_ref, w2_ref, b2_ref, w3_ref, b3_ref, h_ref):
    h1 = jnp.maximum(g1_ref[...] + g2_ref[...], 0.0)
    h2 = jnp.maximum(jnp.dot(h1, w2_ref[...], precision=_PREC) + b2_ref[...],
                     0.0)
    h3 = jnp.dot(h2, w3_ref[...], precision=_PREC) + b3_ref[...]
    h_ref[...] = jnp.maximum(h3, 0.0)


def _edge_mlp2_body(g1_ref, g2_ref, w2_ref, b2_ref, h_ref):
    h1 = jnp.maximum(g1_ref[...] + g2_ref[...], 0.0)
    h_ref[...] = jnp.maximum(
        jnp.dot(h1, w2_ref[...], precision=_PREC) + b2_ref[...], 0.0)


_TE = 2000


def _edge_mlp(g1, g2, w2, b2, w3, b3):
    d3 = w3.shape[1]
    grid = (N_EDGES // _TE,)
    row_spec = pl.BlockSpec((_TE, BIG), lambda i: (i, 0))
    full = lambda a: pl.BlockSpec(a.shape, lambda i: (0,) * a.ndim)
    return pl.pallas_call(
        _edge_mlp_body,
        grid=grid,
        in_specs=[row_spec, row_spec, full(w2), full(b2), full(w3), full(b3)],
        out_specs=pl.BlockSpec((_TE, d3), lambda i: (i, 0)),
        out_shape=_sds((N_EDGES, d3)),
    )(g1, g2, w2, b2, w3, b3)


def _edge_mlp_last(g1, g2, w2, b2):
    grid = (N_EDGES // _TE,)
    row_spec = pl.BlockSpec((_TE, BIG), lambda i: (i, 0))
    full = lambda a: pl.BlockSpec(a.shape, lambda i: (0,) * a.ndim)
    return pl.pallas_call(
        _edge_mlp2_body,
        grid=grid,
        in_specs=[row_spec, row_spec, full(w2), full(b2)],
        out_specs=pl.BlockSpec((_TE, BIG), lambda i: (i, 0)),
        out_shape=_sds((N_EDGES, BIG)),
    )(g1, g2, w2, b2)


def _final_body(s_ref, inv_ref, msk_ref, w3_ref, b3_ref, o_ref):
    y = (s_ref[0, :N_NODES, :] + s_ref[1, :N_NODES, :]) * inv_ref[...]
    o = jnp.dot(y, w3_ref[...], precision=_PREC) + b3_ref[...]
    o_ref[...] = o * msk_ref[...]


def _final(seg, inv, msk, w3, b3):
    return pl.pallas_call(
        _final_body,
        out_shape=_sds((N_NODES, F_IN)),
    )(seg, inv, msk, w3, b3)


def kernel(x, edge_index, params):
    src = edge_index[0].astype(jnp.int32)
    dst = edge_index[1].astype(jnp.int32)
    dst3 = dst.reshape(NW, NCHUNK, CHUNK)

    mlps = params["mlps"]
    wds, wss, b1s = [], [], []
    for layers in mlps:
        w1, b1 = layers[0]
        f = w1.shape[0] // 2
        wds.append(w1[:f] - w1[f:])
        wss.append(w1[f:])
        b1s.append(b1.reshape(1, -1))
    w2s = [layers[1][0] for layers in mlps]
    b2s = [layers[1][1].reshape(1, -1) for layers in mlps]
    w3s, b3s = [], []
    for layers in mlps:
        w3, b3 = layers[2]
        if w3.shape[1] < BIG:
            w3 = jnp.pad(w3, ((0, 0), (0, BIG - w3.shape[1])))
            b3 = jnp.pad(b3, (0, BIG - b3.shape[0]))
        w3s.append(w3)
        b3s.append(b3.reshape(1, -1))

    zeros_big = jnp.zeros((N_PAD, BIG), jnp.float32)
    ones_chunk = jnp.ones((CHUNK, BIG), jnp.float32)

    cnts = _make_sc_counts()(dst3, ones_chunk, zeros_big)
    inv, msk = _prep_cnt(cnts)

    gamma = params["gamma"].reshape(1, -1)
    beta = params["beta"].reshape(1, -1)
    ta, tb = _bn_node(x, gamma, beta, wds[0], wss[0], b1s[0])

    for i in range(6):
        g1, g2 = _make_sc_gather()(ta, tb, src, dst)
        if i == 5:
            h = _edge_mlp_last(g1, g2, w2s[i], b2s[i])
            seg = _make_sc_scatter(BIG)(h, dst3, zeros_big)
            return _final(seg, inv, msk, w3s[i], b3s[i])
        h = _edge_mlp(g1, g2, w2s[i], b2s[i], w3s[i], b3s[i])
        seg = _make_sc_scatter(BIG)(h, dst3, zeros_big)
        ta, tb = _comb_node(seg, inv, wds[i + 1], wss[i + 1], b1s[i + 1])


# speedup vs baseline: 2.9850x; 1.3388x over previous
import functools

import jax
import jax.numpy as jnp
from jax import lax
from jax.experimental import pallas as pl
from jax.experimental.pallas import tpu as pltpu
from jax.experimental.pallas import tpu_sc as plsc

N_NODES = 10000
N_EDGES = 320000
F_IN = 128
BIG = 128
HID = 32
EPS = 1e-5

NC = 2
NS = 16
NW = NC * NS
CHUNK = 80
EPW = N_EDGES // NW
NCHUNK = EPW // CHUNK
CHUNK_S = 40
NCHUNK_S = EPW // CHUNK_S
N_PAD = 10240
ROWS_PER_SUB = N_PAD // NS

_PREC = lax.Precision.HIGHEST


def _sds(shape, dtype=jnp.float32):
    return jax.ShapeDtypeStruct(shape, dtype)


def _vmesh():
    return plsc.VectorSubcoreMesh(core_axis_name="c", subcore_axis_name="s")


NBUF = 5


def _fire(src, dst, sem, add=False):
    pltpu.make_async_copy(src, dst, sem).start(add=add)


def _drain(src, dst, sem):
    pltpu.make_async_copy(src, dst, sem).wait()


@functools.lru_cache(maxsize=None)
def _make_sc_gather():
    @functools.partial(
        pl.kernel,
        out_type=(_sds((N_EDGES, BIG)), _sds((N_EDGES, BIG))),
        mesh=_vmesh(),
        scratch_types=[
            pltpu.VMEM((EPW,), jnp.int32),
            pltpu.VMEM((EPW,), jnp.int32),
            pltpu.VMEM((NBUF, CHUNK, BIG), jnp.float32),
            pltpu.VMEM((NBUF, CHUNK, BIG), jnp.float32),
            pltpu.SemaphoreType.DMA((NBUF,)),
            pltpu.SemaphoreType.DMA((NBUF,)),
            pltpu.SemaphoreType.DMA((NBUF,)),
            pltpu.SemaphoreType.DMA((NBUF,)),
        ],
    )
    def _sc_gather(ta_hbm, tb_hbm, src_hbm, dst_hbm, g1_hbm, g2_hbm,
                   idxs_v, idxd_v, bufa, bufb, gsa, gsb, wsa, wsb):
        w = lax.axis_index("s") * NC + lax.axis_index("c")
        base = w * EPW
        pltpu.sync_copy(src_hbm.at[pl.ds(base, EPW)], idxs_v)
        pltpu.sync_copy(dst_hbm.at[pl.ds(base, EPW)], idxd_v)

        def fire_gather(j, slot):
            off = j * CHUNK
            _fire(ta_hbm.at[idxd_v.at[pl.ds(off, CHUNK)]], bufa.at[slot],
                  gsa.at[slot])
            _fire(tb_hbm.at[idxs_v.at[pl.ds(off, CHUNK)]], bufb.at[slot],
                  gsb.at[slot])

        def wait_gather(j, slot):
            off = j * CHUNK
            _drain(ta_hbm.at[idxd_v.at[pl.ds(off, CHUNK)]], bufa.at[slot],
                   gsa.at[slot])
            _drain(tb_hbm.at[idxs_v.at[pl.ds(off, CHUNK)]], bufb.at[slot],
                   gsb.at[slot])

        def fire_write(j, slot):
            e0 = base + j * CHUNK
            _fire(bufa.at[slot], g1_hbm.at[pl.ds(e0, CHUNK)], wsa.at[slot])
            _fire(bufb.at[slot], g2_hbm.at[pl.ds(e0, CHUNK)], wsb.at[slot])

        def wait_write(j, slot):
            e0 = base + j * CHUNK
            _drain(bufa.at[slot], g1_hbm.at[pl.ds(e0, CHUNK)], wsa.at[slot])
            _drain(bufb.at[slot], g2_hbm.at[pl.ds(e0, CHUNK)], wsb.at[slot])

        fire_gather(0, 0)
        fire_gather(1, 1)

        @pl.loop(0, NCHUNK, step=NBUF)
        def _(g):
            for b in range(NBUF):
                j = g + b
                look = j + 2
                slot2 = (b + 2) % NBUF

                @pl.when(look < NCHUNK)
                def _():
                    @pl.when(look >= NBUF)
                    def _():
                        wait_write(look - NBUF, slot2)
                    fire_gather(look, slot2)

                wait_gather(j, b)
                fire_write(j, b)

        for b in range(NBUF):
            j = NCHUNK - NBUF + b
            wait_write(j, j % NBUF)

    return _sc_gather


@functools.lru_cache(maxsize=None)
def _make_sc_scatter(d):
    @functools.partial(
        pl.kernel,
        out_type=_sds((NC, N_PAD, d)),
        mesh=_vmesh(),
        scratch_types=[
            pltpu.VMEM((NBUF, CHUNK_S), jnp.int32),
            pltpu.VMEM((NBUF, CHUNK_S, d), jnp.float32),
            pltpu.VMEM_SHARED((N_PAD, d), jnp.float32),
            pltpu.SemaphoreType.DMA((NBUF,)),
            pltpu.SemaphoreType.DMA((NBUF,)),
            pltpu.SemaphoreType.DMA((NBUF,)),
        ],
    )
    def scatter_k(h_hbm, dst3_hbm, zeros_hbm, out_hbm, idx_v, buf, acc,
                  isem, rs, ss):
        c = lax.axis_index("c")
        s = lax.axis_index("s")
        pltpu.sync_copy(zeros_hbm.at[pl.ds(s * ROWS_PER_SUB, ROWS_PER_SUB)],
                        acc.at[pl.ds(s * ROWS_PER_SUB, ROWS_PER_SUB)])
        w = c * NS + s
        base = w * EPW
        plsc.subcore_barrier()

        def fire_read(j, slot):
            _fire(h_hbm.at[pl.ds(base + j * CHUNK_S, CHUNK_S)], buf.at[slot],
                  rs.at[slot])
            _fire(dst3_hbm.at[w, j], idx_v.at[slot], isem.at[slot])

        def wait_read(j, slot):
            _drain(h_hbm.at[pl.ds(base + j * CHUNK_S, CHUNK_S)], buf.at[slot],
                   rs.at[slot])
            _drain(dst3_hbm.at[w, j], idx_v.at[slot], isem.at[slot])

        def fire_add(j, slot):
            _fire(buf.at[slot], acc.at[idx_v.at[slot]], ss.at[slot], add=True)

        def wait_add(j, slot):
            _drain(buf.at[slot], acc.at[idx_v.at[slot]], ss.at[slot])

        fire_read(0, 0)
        fire_read(1, 1)

        @pl.loop(0, NCHUNK_S, step=NBUF)
        def _(g):
            for b in range(NBUF):
                j = g + b
                look = j + 2
                slot2 = (b + 2) % NBUF

                @pl.when(look < NCHUNK_S)
                def _():
                    @pl.when(look >= NBUF)
                    def _():
                        wait_add(look - NBUF, slot2)
                    fire_read(look, slot2)

                wait_read(j, b)
                fire_add(j, b)

        for b in range(NBUF):
            wait_add(NCHUNK_S - NBUF + b, b)

        plsc.subcore_barrier()
        pltpu.sync_copy(acc.at[pl.ds(s * ROWS_PER_SUB, ROWS_PER_SUB)],
                        out_hbm.at[c, pl.ds(s * ROWS_PER_SUB, ROWS_PER_SUB)])

    return scatter_k


@functools.lru_cache(maxsize=None)
def _make_sc_counts():
    @functools.partial(
        pl.kernel,
        out_type=_sds((NC, N_PAD, BIG)),
        mesh=_vmesh(),
        scratch_types=[
            pltpu.VMEM((NBUF, CHUNK_S), jnp.int32),
            pltpu.VMEM((CHUNK_S, BIG), jnp.float32),
            pltpu.VMEM_SHARED((N_PAD, BIG), jnp.float32),
            pltpu.SemaphoreType.DMA((NBUF,)),
            pltpu.SemaphoreType.DMA((NBUF,)),
        ],
    )
    def _sc_counts(dst3_hbm, ones_hbm, zeros_hbm, out_hbm, idx_v, buf, acc,
                   isem, ss):
        c = lax.axis_index("c")
        s = lax.axis_index("s")
        pltpu.sync_copy(zeros_hbm.at[pl.ds(s * ROWS_PER_SUB, ROWS_PER_SUB)],
                        acc.at[pl.ds(s * ROWS_PER_SUB, ROWS_PER_SUB)])
        w = c * NS + s
        pltpu.sync_copy(ones_hbm, buf)
        plsc.subcore_barrier()

        def fire_read(j, slot):
            _fire(dst3_hbm.at[w, j], idx_v.at[slot], isem.at[slot])

        def wait_read(j, slot):
            _drain(dst3_hbm.at[w, j], idx_v.at[slot], isem.at[slot])

        def fire_add(j, slot):
            _fire(buf, acc.at[idx_v.at[slot]], ss.at[slot], add=True)

        def wait_add(j, slot):
            _drain(buf, acc.at[idx_v.at[slot]], ss.at[slot])

        fire_read(0, 0)
        fire_read(1, 1)

        @pl.loop(0, NCHUNK_S, step=NBUF)
        def _(g):
            for b in range(NBUF):
                j = g + b
                look = j + 2
                slot2 = (b + 2) % NBUF

                @pl.when(look < NCHUNK_S)
                def _():
                    @pl.when(look >= NBUF)
                    def _():
                        wait_add(look - NBUF, slot2)
                    fire_read(look, slot2)

                wait_read(j, b)
                fire_add(j, b)

        for b in range(NBUF):
            wait_add(NCHUNK_S - NBUF + b, b)

        plsc.subcore_barrier()
        pltpu.sync_copy(acc.at[pl.ds(s * ROWS_PER_SUB, ROWS_PER_SUB)],
                        out_hbm.at[c, pl.ds(s * ROWS_PER_SUB, ROWS_PER_SUB)])

    return _sc_counts


def _bn_node_body(x_ref, g_ref, be_ref, wd_ref, ws_ref, b1_ref,
                  ta_ref, tb_ref):
    x = x_ref[...]
    mu = jnp.mean(x, axis=0, keepdims=True)
    var = jnp.mean((x - mu) ** 2, axis=0, keepdims=True)
    y = (x - mu) * (1.0 / jnp.sqrt(var + EPS)) * g_ref[...] + be_ref[...]
    ta_ref[...] = jnp.dot(y, wd_ref[...], precision=_PREC) + b1_ref[...]
    tb_ref[...] = jnp.dot(y, ws_ref[...], precision=_PREC)


def _bn_node(x, gamma, beta, wd, ws, b1):
    return pl.pallas_call(
        _bn_node_body,
        out_shape=(_sds((N_NODES, BIG)), _sds((N_NODES, BIG))),
    )(x, gamma, beta, wd, ws, b1)


def _prep_cnt_body(c_ref, inv_ref, msk_ref):
    cnt = c_ref[0, :N_NODES, 0:1] + c_ref[1, :N_NODES, 0:1]
    inv_ref[...] = 1.0 / jnp.maximum(cnt, 1.0)
    msk_ref[...] = jnp.where(cnt > 0.0, 1.0, 0.0)


def _prep_cnt(cnts):
    return pl.pallas_call(
        _prep_cnt_body,
        out_shape=(_sds((N_NODES, 1)), _sds((N_NODES, 1))),
    )(cnts)


def _comb_node_body(s_ref, inv_ref, wd_ref, ws_ref, b1_ref, ta_ref, tb_ref):
    f = wd_ref.shape[0]
    y = (s_ref[0, :N_NODES, :f] + s_ref[1, :N_NODES, :f]) * inv_ref[...]
    ta_ref[...] = jnp.dot(y, wd_ref[...], precision=_PREC) + b1_ref[...]
    tb_ref[...] = jnp.dot(y, ws_ref[...], precision=_PREC)


def _comb_node(seg, inv, wd, ws, b1):
    return pl.pallas_call(
        _comb_node_body,
        out_shape=(_sds((N_NODES, BIG)), _sds((N_NODES, BIG))),
    )(seg, inv, wd, ws, b1)


def _edge_mlp_body(g1_ref, g2_ref, w2_ref, b2_ref, w3_ref, b3_ref, h_ref):
    h1 = jnp.maximum(g1_ref[...] + g2_ref[...], 0.0)
    h2 = jnp.maximum(jnp.dot(h1, w2_ref[...], precision=_PREC) + b2_ref[...],
                     0.0)
    h3 = jnp.dot(h2, w3_ref[...], precision=_PREC) + b3_ref[...]
    h_ref[...] = jnp.maximum(h3, 0.0)


def _edge_mlp2_body(g1_ref, g2_ref, w2_ref, b2_ref, h_ref):
    h1 = jnp.maximum(g1_ref[...] + g2_ref[...], 0.0)
    h_ref[...] = jnp.maximum(
        jnp.dot(h1, w2_ref[...], precision=_PREC) + b2_ref[...], 0.0)


_TE = 2000


def _edge_mlp(g1, g2, w2, b2, w3, b3):
    d3 = w3.shape[1]
    grid = (N_EDGES // _TE,)
    row_spec = pl.BlockSpec((_TE, BIG), lambda i: (i, 0))
    full = lambda a: pl.BlockSpec(a.shape, lambda i: (0,) * a.ndim)
    return pl.pallas_call(
        _edge_mlp_body,
        grid=grid,
        in_specs=[row_spec, row_spec, full(w2), full(b2), full(w3), full(b3)],
        out_specs=pl.BlockSpec((_TE, d3), lambda i: (i, 0)),
        out_shape=_sds((N_EDGES, d3)),
    )(g1, g2, w2, b2, w3, b3)


def _edge_mlp_last(g1, g2, w2, b2):
    grid = (N_EDGES // _TE,)
    row_spec = pl.BlockSpec((_TE, BIG), lambda i: (i, 0))
    full = lambda a: pl.BlockSpec(a.shape, lambda i: (0,) * a.ndim)
    return pl.pallas_call(
        _edge_mlp2_body,
        grid=grid,
        in_specs=[row_spec, row_spec, full(w2), full(b2)],
        out_specs=pl.BlockSpec((_TE, BIG), lambda i: (i, 0)),
        out_shape=_sds((N_EDGES, BIG)),
    )(g1, g2, w2, b2)


def _final_body(s_ref, inv_ref, msk_ref, w3_ref, b3_ref, o_ref):
    y = (s_ref[0, :N_NODES, :] + s_ref[1, :N_NODES, :]) * inv_ref[...]
    o = jnp.dot(y, w3_ref[...], precision=_PREC) + b3_ref[...]
    o_ref[...] = o * msk_ref[...]


def _final(seg, inv, msk, w3, b3):
    return pl.pallas_call(
        _final_body,
        out_shape=_sds((N_NODES, F_IN)),
    )(seg, inv, msk, w3, b3)


def kernel(x, edge_index, params):
    src = edge_index[0].astype(jnp.int32)
    dst = edge_index[1].astype(jnp.int32)
    dst3 = dst.reshape(NW, NCHUNK_S, CHUNK_S)

    mlps = params["mlps"]
    wds, wss, b1s = [], [], []
    for layers in mlps:
        w1, b1 = layers[0]
        f = w1.shape[0] // 2
        wds.append(w1[:f] - w1[f:])
        wss.append(w1[f:])
        b1s.append(b1.reshape(1, -1))
    w2s = [layers[1][0] for layers in mlps]
    b2s = [layers[1][1].reshape(1, -1) for layers in mlps]
    w3s, b3s = [], []
    for layers in mlps:
        w3, b3 = layers[2]
        if w3.shape[1] < BIG:
            w3 = jnp.pad(w3, ((0, 0), (0, BIG - w3.shape[1])))
            b3 = jnp.pad(b3, (0, BIG - b3.shape[0]))
        w3s.append(w3)
        b3s.append(b3.reshape(1, -1))

    zeros_big = jnp.zeros((N_PAD, BIG), jnp.float32)
    ones_chunk = jnp.ones((CHUNK_S, BIG), jnp.float32)

    cnts = _make_sc_counts()(dst3, ones_chunk, zeros_big)
    inv, msk = _prep_cnt(cnts)

    gamma = params["gamma"].reshape(1, -1)
    beta = params["beta"].reshape(1, -1)
    ta, tb = _bn_node(x, gamma, beta, wds[0], wss[0], b1s[0])

    for i in range(6):
        g1, g2 = _make_sc_gather()(ta, tb, src, dst)
        if i == 5:
            h = _edge_mlp_last(g1, g2, w2s[i], b2s[i])
            seg = _make_sc_scatter(BIG)(h, dst3, zeros_big)
            return _final(seg, inv, msk, w3s[i], b3s[i])
        h = _edge_mlp(g1, g2, w2s[i], b2s[i], w3s[i], b3s[i])
        seg = _make_sc_scatter(BIG)(h, dst3, zeros_big)
        ta, tb = _comb_node(seg, inv, wds[i + 1], wss[i + 1], b1s[i + 1])
